# baseline (speedup 1.0000x reference)
import functools

import jax
import jax.numpy as jnp
from jax import lax
from jax.experimental import pallas as pl
from jax.experimental.pallas import tpu as pltpu
from jax.experimental.pallas import tpu_sc as plsc

B, S, D = 4096, 200, 256
NTYPE, NIDX = 5, 50
NCOMBO = NTYPE * NIDX
NTAB = NCOMBO * S
NC, NS = 2, 16
NW = NC * NS
NB = B // NW
LANES = 16
G0, G1 = 104, 96
MSTR = 208
PAGE_LO, PAGE_HI = NIDX * S, 2 * NIDX * S


def _style_mask_body(types_ref, inds_ref, sv_ref, w1_ref, b1_ref, w2_ref,
                     b2_ref, styled_ref, mask_ref, gidx_ref):
    h = jnp.dot(sv_ref[...], w1_ref[...], preferred_element_type=jnp.float32)
    h = jnp.maximum(h + b1_ref[...][None, :], 0.0)
    styled = jnp.dot(h, w2_ref[...], preferred_element_type=jnp.float32)
    styled_ref[...] = styled + b2_ref[...][None, :]
    mask_ref[...] = types_ref[...] == 0
    s_iota = lax.broadcasted_iota(jnp.int32, (B, S), 1)
    gidx_ref[...] = (types_ref[...] * NIDX + inds_ref[...]) * S + s_iota


def _tc_pre(types, inds, style_vector, w1, b1, w2, b2):
    return pl.pallas_call(
        _style_mask_body,
        out_shape=[
            jax.ShapeDtypeStruct((B, D), jnp.float32),
            jax.ShapeDtypeStruct((B, S), jnp.bool_),
            jax.ShapeDtypeStruct((B, S), jnp.int32),
        ],
    )(types, inds, style_vector, w1, b1, w2, b2)


def _bigtab_body(temb_ref, iemb_ref, pemb_ref, out_ref):
    g = pl.program_id(0)
    trow = jnp.zeros((1, D), jnp.float32)
    for t in range(NTYPE):
        trow = trow + temb_ref[pl.ds(t, 1), :] * jnp.where(g == t, 1.0, 0.0)
    big = (iemb_ref[...][:, None, :] + pemb_ref[...][None, :, :]
           + trow[None, :, :])
    out_ref[...] = big.reshape(NIDX * S, D)


def _tc_bigtab(temb, iemb, pemb):
    return pl.pallas_call(
        _bigtab_body,
        grid=(NTYPE,),
        in_specs=[
            pl.BlockSpec((NTYPE, D), lambda g: (0, 0)),
            pl.BlockSpec((NIDX, D), lambda g: (0, 0)),
            pl.BlockSpec((S, D), lambda g: (0, 0)),
        ],
        out_specs=pl.BlockSpec((NIDX * S, D), lambda g: (g, 0)),
        out_shape=jax.ShapeDtypeStruct((NTAB, D), jnp.float32),
    )(temb, iemb, pemb)


@functools.partial(
    pl.kernel,
    out_type=jax.ShapeDtypeStruct((B * S, D), jnp.float32),
    mesh=plsc.VectorSubcoreMesh(
        core_axis_name="c", subcore_axis_name="s",
        num_cores=NC, num_subcores=NS),
    compiler_params=pltpu.CompilerParams(needs_layout_passes=False),
    scratch_types=[
        pltpu.VMEM((104, D), jnp.float32),
        pltpu.VMEM((96, D), jnp.float32),
        pltpu.VMEM((104, D), jnp.float32),
        pltpu.VMEM((96, D), jnp.float32),
        pltpu.VMEM((3 * MSTR,), jnp.int32),
        pltpu.VMEM((3 * D,), jnp.float32),
        pltpu.VMEM((MSTR + LANES,), jnp.int32),
        pltpu.SemaphoreType.DMA,
        pltpu.SemaphoreType.DMA,
        pltpu.SemaphoreType.DMA,
        pltpu.SemaphoreType.DMA,
        pltpu.SemaphoreType.DMA,
        pltpu.SemaphoreType.DMA,
        pltpu.SemaphoreType.DMA,
        pltpu.SemaphoreType.DMA,
        pltpu.SemaphoreType.DMA,
    ],
)
def _sc_embed(gidx_hbm, styled_hbm, tab_hbm, out_hbm,
              blk0, blk1, blk2, blk3, gidxb, styb, pglist,
              gsem0, gsem1, gsem2, gsem3, osem0, osem1, osem2, osem3, fsem):
    blks = (blk0, blk1, blk2, blk3)
    gsems = (gsem0, gsem1, gsem2, gsem3)
    osems = (osem0, osem1, osem2, osem3)
    HS = (104, 96)
    wid = lax.axis_index("s") * NC + lax.axis_index("c")
    b0 = wid * NB
    iota = lax.iota(jnp.int32, LANES)

    def meta_src(r):
        return (gidx_hbm.at[pl.ds((b0 + r) * S, S)],
                styled_hbm.at[pl.ds((b0 + r) * D, D)])

    def meta_dst(r):
        m = (r % 3)
        return (gidxb.at[pl.ds(m * MSTR, S)], styb.at[pl.ds(m * D, D)])

    def gather_sd(r, h, blk):
        m = (r % 3) * MSTR
        return (tab_hbm.at[gidxb.at[pl.ds(m + h * 104, HS[h])]], blk)

    for r in (0, 1):
        for sx, dx in zip(meta_src(r), meta_dst(r)):
            pltpu.sync_copy(sx, dx)
    for h in (0, 1):
        sx, dx = gather_sd(0, h, blks[h])
        pltpu.async_copy(sx, dx, gsems[h])

    def do_step(k, r, h, p):
        mb = (r % 3)
        blk_p = blks[p]

        if h == 0:
            @pl.when((r >= 1) & (r + 1 < NB))
            def _():
                for sx, dx in zip(meta_src(r + 1), meta_dst(r + 1)):
                    pltpu.make_async_copy(sx, dx, fsem).wait()

            @pl.when(r + 2 < NB)
            def _():
                for sx, dx in zip(meta_src(r + 2), meta_dst(r + 2)):
                    pltpu.async_copy(sx, dx, fsem)

        sx, dx = gather_sd(r, h, blk_p)
        pltpu.make_async_copy(sx, dx, gsems[p]).wait()

        sty = tuple(styb[pl.ds(mb * D + kk * LANES, LANES)]
                    for kk in range(D // LANES))
        hs = HS[h]
        cnt = 0
        for w in range((hs + LANES - 1) // LANES):
            off = w * LANES
            gv = gidxb[pl.ds(mb * MSTR + h * 104 + off, LANES)]
            pm = (gv >= PAGE_LO) & (gv < PAGE_HI)
            if off + LANES > hs:
                pm = pm & (iota < hs - off)
            plsc.store_compressed(pglist.at[pl.ds(cnt, LANES)],
                                  iota + off, mask=pm)
            cnt = cnt + plsc.all_reduce_population_count(pm)[0]

        def fix_body(wi, carry):
            pg = pglist[pl.ds(wi * LANES, LANES)]
            for l in range(LANES):
                s_l = pg[l]

                @pl.when(wi * LANES + l < cnt)
                def _(_s=s_l):
                    for kk in range(D // LANES):
                        blk_p[_s, pl.ds(kk * LANES, LANES)] += sty[kk]
            return carry

        lax.fori_loop(0, (cnt + LANES - 1) // LANES, fix_body, 0)

        pltpu.async_copy(
            blk_p, out_hbm.at[pl.ds((b0 + r) * S + 104 * h, HS[h])],
            osems[p])

        p2 = (p + 2) % 4

        @pl.when(k >= 2)
        def _():
            pltpu.make_async_copy(
                blks[p2],
                out_hbm.at[pl.ds((b0 + r - 1) * S + 104 * h, HS[h])],
                osems[p2]).wait()

        @pl.when(k + 2 < 2 * NB)
        def _():
            r2 = r + 1
            sx2, dx2 = gather_sd(r2, h, blks[p2])
            pltpu.async_copy(sx2, dx2, gsems[p2])

    def quad_body(u, carry):
        do_step(4 * u + 0, 2 * u, 0, 0)
        do_step(4 * u + 1, 2 * u, 1, 1)
        do_step(4 * u + 2, 2 * u + 1, 0, 2)
        do_step(4 * u + 3, 2 * u + 1, 1, 3)
        return carry

    lax.fori_loop(0, NB // 2, quad_body, 0)

    for p2, h in ((2, 0), (3, 1)):
        pltpu.make_async_copy(
            blks[p2],
            out_hbm.at[pl.ds((b0 + NB - 1) * S + 104 * h, HS[h])],
            osems[p2]).wait()


def kernel(element_types, element_indices, style_vector, type_emb, idx_emb,
           W1, b1, W2, b2, pos_emb):
    types = element_types.astype(jnp.int32)
    inds = element_indices.astype(jnp.int32)
    styled, mask, gidx = _tc_pre(types, inds, style_vector, W1, b1, W2, b2)
    bigtab = _tc_bigtab(type_emb, idx_emb, pos_emb)
    final = _sc_embed(gidx.reshape(-1), styled.reshape(-1), bigtab)
    return final.reshape(B, S, D), mask
